# baseline (device time: 47185 ns/iter reference)
import jax
import jax.numpy as jnp
from jax import lax
from jax.experimental import pallas as pl
from jax.experimental.pallas import tpu as pltpu

N_DEV = 4


def kernel(x, w_mat):
    m_per, k = x.shape
    _, n_per = w_mat.shape

    def body(x_ref, w_ref, out_ref, comm_ref, send_sems, recv_sems):
        my_pos = lax.axis_index("i")
        left = lax.rem(my_pos + N_DEV - 1, N_DEV)
        right = lax.rem(my_pos + 1, N_DEV)

        barrier_sem = pltpu.get_barrier_semaphore()
        for nbr in (left, right):
            pl.semaphore_signal(
                barrier_sem, inc=1,
                device_id=(nbr,), device_id_type=pl.DeviceIdType.MESH,
            )
        pl.semaphore_wait(barrier_sem, 2)

        comm_ref[0] = x_ref[:]

        out_ref[pl.ds(my_pos * m_per, m_per), :] = jnp.maximum(
            jnp.dot(x_ref[:], w_ref[:], preferred_element_type=jnp.float32),
            0.0,
        )

        for h in range(1, N_DEV):
            rdma = pltpu.make_async_remote_copy(
                src_ref=comm_ref.at[h - 1],
                dst_ref=comm_ref.at[h],
                send_sem=send_sems.at[h - 1],
                recv_sem=recv_sems.at[h - 1],
                device_id=(right,),
                device_id_type=pl.DeviceIdType.MESH,
            )
            rdma.start()
            rdma.wait()

            origin = lax.rem(my_pos + N_DEV - h, N_DEV)
            out_ref[pl.ds(origin * m_per, m_per), :] = jnp.maximum(
                jnp.dot(
                    comm_ref[h], w_ref[:],
                    preferred_element_type=jnp.float32,
                ),
                0.0,
            )

    return pl.pallas_call(
        body,
        out_shape=jax.ShapeDtypeStruct((N_DEV * m_per, n_per), jnp.float32),
        in_specs=[
            pl.BlockSpec(memory_space=pltpu.VMEM),
            pl.BlockSpec(memory_space=pltpu.VMEM),
        ],
        out_specs=pl.BlockSpec(memory_space=pltpu.VMEM),
        scratch_shapes=[
            pltpu.VMEM((N_DEV, m_per, k), jnp.float32),
            pltpu.SemaphoreType.DMA((N_DEV - 1,)),
            pltpu.SemaphoreType.DMA((N_DEV - 1,)),
        ],
        compiler_params=pltpu.CompilerParams(collective_id=0),
    )(x, w_mat)


# device time: 29045 ns/iter; 1.6245x vs baseline; 1.6245x over previous
import jax
import jax.numpy as jnp
from jax import lax
from jax.experimental import pallas as pl
from jax.experimental.pallas import tpu as pltpu

N_DEV = 4


def kernel(x, w_mat):
    m_per, k = x.shape
    _, n_per = w_mat.shape
    half = m_per // 2

    def body(x_ref, w_ref, out_ref,
             cw_ref, ccw_ref,
             cw_send, cw_recv, ccw_send, ccw_recv):
        my_pos = lax.axis_index("i")
        left = lax.rem(my_pos + N_DEV - 1, N_DEV)
        right = lax.rem(my_pos + 1, N_DEV)

        barrier_sem = pltpu.get_barrier_semaphore()
        for nbr in (left, right):
            pl.semaphore_signal(
                barrier_sem, inc=1,
                device_id=(nbr,), device_id_type=pl.DeviceIdType.MESH,
            )
        pl.semaphore_wait(barrier_sem, 2)

        cw_ref[0] = x_ref[0:half, :]
        ccw_ref[0] = x_ref[half:m_per, :]

        def hop_rdmas(h):
            cw = pltpu.make_async_remote_copy(
                src_ref=cw_ref.at[h - 1],
                dst_ref=cw_ref.at[h],
                send_sem=cw_send.at[h - 1],
                recv_sem=cw_recv.at[h - 1],
                device_id=(right,),
                device_id_type=pl.DeviceIdType.MESH,
            )
            ccw = pltpu.make_async_remote_copy(
                src_ref=ccw_ref.at[h - 1],
                dst_ref=ccw_ref.at[h],
                send_sem=ccw_send.at[h - 1],
                recv_sem=ccw_recv.at[h - 1],
                device_id=(left,),
                device_id_type=pl.DeviceIdType.MESH,
            )
            return cw, ccw

        def gemm_store(src_ref, slot, origin, row_off):
            out_ref[pl.ds(origin * m_per + row_off, half), :] = jnp.maximum(
                jnp.dot(src_ref[slot], w_ref[:],
                        preferred_element_type=jnp.float32),
                0.0,
            )

        cw1, ccw1 = hop_rdmas(1)
        cw1.start()
        ccw1.start()
        out_ref[pl.ds(my_pos * m_per, m_per), :] = jnp.maximum(
            jnp.dot(x_ref[:], w_ref[:], preferred_element_type=jnp.float32),
            0.0,
        )
        cw1.wait()
        ccw1.wait()

        for h in range(2, N_DEV):
            cwh, ccwh = hop_rdmas(h)
            cwh.start()
            ccwh.start()
            gemm_store(cw_ref, h - 1, lax.rem(my_pos + N_DEV - (h - 1), N_DEV), 0)
            gemm_store(ccw_ref, h - 1, lax.rem(my_pos + (h - 1), N_DEV), half)
            cwh.wait()
            ccwh.wait()

        gemm_store(cw_ref, N_DEV - 1, lax.rem(my_pos + 1, N_DEV), 0)
        gemm_store(ccw_ref, N_DEV - 1, lax.rem(my_pos + N_DEV - 1, N_DEV), half)

    return pl.pallas_call(
        body,
        out_shape=jax.ShapeDtypeStruct((N_DEV * m_per, n_per), jnp.float32),
        in_specs=[
            pl.BlockSpec(memory_space=pltpu.VMEM),
            pl.BlockSpec(memory_space=pltpu.VMEM),
        ],
        out_specs=pl.BlockSpec(memory_space=pltpu.VMEM),
        scratch_shapes=[
            pltpu.VMEM((N_DEV, half, k), jnp.float32),
            pltpu.VMEM((N_DEV, half, k), jnp.float32),
            pltpu.SemaphoreType.DMA((N_DEV - 1,)),
            pltpu.SemaphoreType.DMA((N_DEV - 1,)),
            pltpu.SemaphoreType.DMA((N_DEV - 1,)),
            pltpu.SemaphoreType.DMA((N_DEV - 1,)),
        ],
        compiler_params=pltpu.CompilerParams(collective_id=0),
    )(x, w_mat)


# device time: 25746 ns/iter; 1.8327x vs baseline; 1.1281x over previous
import jax
import jax.numpy as jnp
from jax import lax
from jax.experimental import pallas as pl
from jax.experimental.pallas import tpu as pltpu

N_DEV = 4
N_HOP = N_DEV - 1
P = 2


def kernel(x, w_mat):
    m_per, k = x.shape
    _, n_per = w_mat.shape
    half = m_per // 2
    prows = half // P

    def body(x_ref, w_ref, out_ref,
             cw_ref, ccw_ref,
             cw_send, cw_recv, ccw_send, ccw_recv):
        my_pos = lax.axis_index("i")
        left = lax.rem(my_pos + N_DEV - 1, N_DEV)
        right = lax.rem(my_pos + 1, N_DEV)

        barrier_sem = pltpu.get_barrier_semaphore()
        for nbr in (left, right):
            pl.semaphore_signal(
                barrier_sem, inc=1,
                device_id=(nbr,), device_id_type=pl.DeviceIdType.MESH,
            )
        pl.semaphore_wait(barrier_sem, 2)

        def make_rdma(h, q):
            ds = pl.ds(q * prows, prows)
            cw_src = x_ref.at[ds] if h == 1 else cw_ref.at[h - 2, ds]
            ccw_src = (x_ref.at[pl.ds(half + q * prows, prows)]
                       if h == 1 else ccw_ref.at[h - 2, ds])
            cw = pltpu.make_async_remote_copy(
                src_ref=cw_src,
                dst_ref=cw_ref.at[h - 1, ds],
                send_sem=cw_send.at[h - 1, q],
                recv_sem=cw_recv.at[h - 1, q],
                device_id=(right,),
                device_id_type=pl.DeviceIdType.MESH,
            )
            ccw = pltpu.make_async_remote_copy(
                src_ref=ccw_src,
                dst_ref=ccw_ref.at[h - 1, ds],
                send_sem=ccw_send.at[h - 1, q],
                recv_sem=ccw_recv.at[h - 1, q],
                device_id=(left,),
                device_id_type=pl.DeviceIdType.MESH,
            )
            return cw, ccw

        rdmas = {hq: make_rdma(*hq)
                 for hq in [(h, q) for h in range(1, N_DEV) for q in range(P)]}

        def gemm_store(src_ref, slot, origin, row_off):
            out_ref[pl.ds(origin * m_per + row_off, half), :] = jnp.maximum(
                jnp.dot(src_ref[slot], w_ref[:],
                        preferred_element_type=jnp.float32),
                0.0,
            )

        for q in range(P):
            rdmas[(1, q)][0].start()
            rdmas[(1, q)][1].start()
        out_ref[pl.ds(my_pos * m_per, m_per), :] = jnp.maximum(
            jnp.dot(x_ref[:], w_ref[:], preferred_element_type=jnp.float32),
            0.0,
        )

        for h in range(1, N_DEV):
            for q in range(P):
                cw, ccw = rdmas[(h, q)]
                cw.wait_recv()
                if h < N_HOP:
                    rdmas[(h + 1, q)][0].start()
                ccw.wait_recv()
                if h < N_HOP:
                    rdmas[(h + 1, q)][1].start()
            gemm_store(cw_ref, h - 1, lax.rem(my_pos + N_DEV - h, N_DEV), 0)
            gemm_store(ccw_ref, h - 1, lax.rem(my_pos + h, N_DEV), half)

        for hq in rdmas:
            rdmas[hq][0].wait_send()
            rdmas[hq][1].wait_send()

    return pl.pallas_call(
        body,
        out_shape=jax.ShapeDtypeStruct((N_DEV * m_per, n_per), jnp.float32),
        in_specs=[
            pl.BlockSpec(memory_space=pltpu.VMEM),
            pl.BlockSpec(memory_space=pltpu.VMEM),
        ],
        out_specs=pl.BlockSpec(memory_space=pltpu.VMEM),
        scratch_shapes=[
            pltpu.VMEM((N_HOP, half, k), jnp.float32),
            pltpu.VMEM((N_HOP, half, k), jnp.float32),
            pltpu.SemaphoreType.DMA((N_HOP, P)),
            pltpu.SemaphoreType.DMA((N_HOP, P)),
            pltpu.SemaphoreType.DMA((N_HOP, P)),
            pltpu.SemaphoreType.DMA((N_HOP, P)),
        ],
        compiler_params=pltpu.CompilerParams(collective_id=0),
    )(x, w_mat)


# device time: 25740 ns/iter; 1.8331x vs baseline; 1.0002x over previous
import jax
import jax.numpy as jnp
from jax import lax
from jax.experimental import pallas as pl
from jax.experimental.pallas import tpu as pltpu

N_DEV = 4
P = 2


def kernel(x, w_mat):
    m_per, k = x.shape
    _, n_per = w_mat.shape
    half = m_per // 2
    qrows = half // P

    def body(x_ref, w_ref, out_ref,
             lt_ref, lb_ref, dt_ref,
             rb_ref, rt_ref, db_ref,
             lt_s, lt_r, lb_s, lb_r, dt_s, dt_r,
             rb_s, rb_r, rt_s, rt_r, db_s, db_r):
        my_pos = lax.axis_index("i")
        left = lax.rem(my_pos + N_DEV - 1, N_DEV)
        right = lax.rem(my_pos + 1, N_DEV)

        barrier_sem = pltpu.get_barrier_semaphore()
        for nbr in (left, right):
            pl.semaphore_signal(
                barrier_sem, inc=1,
                device_id=(nbr,), device_id_type=pl.DeviceIdType.MESH,
            )
        pl.semaphore_wait(barrier_sem, 2)

        def rcopy(src, dst, ssem, rsem, dev):
            return pltpu.make_async_remote_copy(
                src_ref=src, dst_ref=dst, send_sem=ssem, recv_sem=rsem,
                device_id=(dev,), device_id_type=pl.DeviceIdType.MESH,
            )

        lt = [rcopy(x_ref.at[pl.ds(q * qrows, qrows)],
                    lt_ref.at[pl.ds(q * qrows, qrows)],
                    lt_s.at[q], lt_r.at[q], right) for q in range(P)]
        lb = rcopy(x_ref.at[pl.ds(half, half)], lb_ref,
                   lb_s.at[0], lb_r.at[0], right)
        rb = [rcopy(x_ref.at[pl.ds(half + q * qrows, qrows)],
                    rb_ref.at[pl.ds(q * qrows, qrows)],
                    rb_s.at[q], rb_r.at[q], left) for q in range(P)]
        rt = rcopy(x_ref.at[pl.ds(0, half)], rt_ref,
                   rt_s.at[0], rt_r.at[0], left)
        dt = [rcopy(lt_ref.at[pl.ds(q * qrows, qrows)],
                    dt_ref.at[pl.ds(q * qrows, qrows)],
                    dt_s.at[q], dt_r.at[q], right) for q in range(P)]
        db = [rcopy(rb_ref.at[pl.ds(q * qrows, qrows)],
                    db_ref.at[pl.ds(q * qrows, qrows)],
                    db_s.at[q], db_r.at[q], left) for q in range(P)]

        for q in range(P):
            lt[q].start()
            rb[q].start()
        lb.start()
        rt.start()

        def gemm_store(data, origin, row_off, rows):
            out_ref[pl.ds(origin * m_per + row_off, rows), :] = jnp.maximum(
                jnp.dot(data, w_ref[:], preferred_element_type=jnp.float32),
                0.0,
            )

        gemm_store(x_ref[:], my_pos, 0, m_per)

        for q in range(P):
            lt[q].wait_recv()
            dt[q].start()
            rb[q].wait_recv()
            db[q].start()

        gemm_store(lt_ref[:], left, 0, half)
        gemm_store(rb_ref[:], right, half, half)

        lb.wait_recv()
        rt.wait_recv()
        gemm_store(lb_ref[:], left, half, half)
        gemm_store(rt_ref[:], right, 0, half)

        diag = lax.rem(my_pos + 2, N_DEV)
        for q in range(P):
            dt[q].wait_recv()
            db[q].wait_recv()
        gemm_store(dt_ref[:], diag, 0, half)
        gemm_store(db_ref[:], diag, half, half)

        for q in range(P):
            lt[q].wait_send()
            rb[q].wait_send()
            dt[q].wait_send()
            db[q].wait_send()
        lb.wait_send()
        rt.wait_send()

    half_buf = pltpu.VMEM((half, k), jnp.float32)
    return pl.pallas_call(
        body,
        out_shape=jax.ShapeDtypeStruct((N_DEV * m_per, n_per), jnp.float32),
        in_specs=[
            pl.BlockSpec(memory_space=pltpu.VMEM),
            pl.BlockSpec(memory_space=pltpu.VMEM),
        ],
        out_specs=pl.BlockSpec(memory_space=pltpu.VMEM),
        scratch_shapes=[
            half_buf, half_buf, half_buf,
            half_buf, half_buf, half_buf,
            pltpu.SemaphoreType.DMA((P,)), pltpu.SemaphoreType.DMA((P,)),
            pltpu.SemaphoreType.DMA((1,)), pltpu.SemaphoreType.DMA((1,)),
            pltpu.SemaphoreType.DMA((P,)), pltpu.SemaphoreType.DMA((P,)),
            pltpu.SemaphoreType.DMA((P,)), pltpu.SemaphoreType.DMA((P,)),
            pltpu.SemaphoreType.DMA((1,)), pltpu.SemaphoreType.DMA((1,)),
            pltpu.SemaphoreType.DMA((P,)), pltpu.SemaphoreType.DMA((P,)),
        ],
        compiler_params=pltpu.CompilerParams(collective_id=0),
    )(x, w_mat)
